# baseline (speedup 1.0000x reference)
import jax
import jax.numpy as jnp
from jax import lax
from jax.experimental import pallas as pl
from jax.experimental.pallas import tpu as pltpu
import jax.experimental.pallas.tpu_sc as plsc

N = 10000
E = 320000
D_IN = 128
D_OUT = 128

NC = 2
NS = 16
NW = NC * NS
EW = E // NW
K = 80
NCHUNK = EW // K
NPAD = 10240
RPT = NPAD // NS
DEGW = 16

_MESH = plsc.VectorSubcoreMesh(
    core_axis_name="c", subcore_axis_name="s", num_cores=NC, num_subcores=NS
)


def _deg_body(rows_hbm, ones_hbm, zeros_hbm, out_hbm, rowv, onesv, acc):
    c = lax.axis_index("c")
    s = lax.axis_index("s")
    g = c * NS + s
    pltpu.sync_copy(rows_hbm.at[g], rowv)
    pltpu.sync_copy(ones_hbm, onesv)
    sl = pl.ds(s * RPT, RPT)
    pltpu.sync_copy(zeros_hbm.at[sl], acc.at[sl])
    plsc.subcore_barrier()

    def step(j, carry):
        pltpu.sync_copy(onesv, acc.at[rowv.at[j]], add=True)
        return carry

    lax.fori_loop(0, NCHUNK, step, 0)
    plsc.subcore_barrier()
    pltpu.sync_copy(acc.at[sl], out_hbm.at[c, sl])


def _edge_body(y_hbm, rows_hbm, cols_hbm, zeros_hbm, out_hbm,
               rowv, colv, bufa, bufb, acc, ga, gb, sa, sb):
    c = lax.axis_index("c")
    s = lax.axis_index("s")
    g = c * NS + s
    pltpu.sync_copy(rows_hbm.at[g], rowv)
    pltpu.sync_copy(cols_hbm.at[pl.ds(g * EW, EW)], colv)
    sl = pl.ds(s * RPT, RPT)
    pltpu.sync_copy(zeros_hbm.at[sl], acc.at[sl])
    plsc.subcore_barrier()

    def gather(j, buf, sem):
        pltpu.async_copy(y_hbm.at[colv.at[pl.ds(j * K, K)]], buf, sem)

    def gwait(j, buf, sem):
        pltpu.make_async_copy(y_hbm.at[colv.at[pl.ds(j * K, K)]], buf, sem).wait()

    def scat(j, buf, sem):
        pltpu.async_copy(buf, acc.at[rowv.at[j]], sem, add=True)

    def swait(j, buf, sem):
        pltpu.make_async_copy(buf, acc.at[rowv.at[j]], sem).wait()

    gather(0, bufa, ga)
    gather(1, bufb, gb)

    def pair(i, carry):
        ja = 2 * i
        jb = 2 * i + 1
        gwait(ja, bufa, ga)
        scat(ja, bufa, sa)
        gwait(jb, bufb, gb)
        swait(ja, bufa, sa)
        scat(jb, bufb, sb)
        gather(ja + 2, bufa, ga)
        swait(jb, bufb, sb)
        gather(jb + 2, bufb, gb)
        return carry

    lax.fori_loop(0, (NCHUNK - 3) // 2, pair, 0)
    j2, j3, j4 = NCHUNK - 3, NCHUNK - 2, NCHUNK - 1
    gwait(j2, bufa, ga)
    scat(j2, bufa, sa)
    gwait(j3, bufb, gb)
    swait(j2, bufa, sa)
    scat(j3, bufb, sb)
    gather(j4, bufa, ga)
    gwait(j4, bufa, ga)
    swait(j3, bufb, sb)
    scat(j4, bufa, sa)
    swait(j4, bufa, sa)

    plsc.subcore_barrier()
    pltpu.sync_copy(acc.at[sl], out_hbm.at[c, sl])


def _pre_body(x_ref, w_ref, dp_ref, xw_ref, y_ref):
    xw = lax.dot_general(
        x_ref[...], w_ref[...], (((1,), (1,)), ((), ())),
        preferred_element_type=jnp.float32)
    deg = dp_ref[0, :, 0:1] + dp_ref[1, :, 0:1] + 1.0
    xw_ref[...] = xw
    y_ref[...] = xw * (1.0 / jnp.sqrt(deg))


def _post_body(ap_ref, dp_ref, xw_ref, o_ref):
    deg = dp_ref[0, :, 0:1] + dp_ref[1, :, 0:1] + 1.0
    dis = 1.0 / jnp.sqrt(deg)
    o_ref[...] = dis * (ap_ref[0] + ap_ref[1]) + xw_ref[...] / deg


_R = 400


@jax.jit
def kernel(x, edge_index, W):
    row = edge_index[0].astype(jnp.int32)
    col = edge_index[1].astype(jnp.int32)
    rows3 = row.reshape(NW, NCHUNK, K)
    ones16 = jnp.ones((K, DEGW), jnp.float32)
    zeros16 = jnp.zeros((NPAD, DEGW), jnp.float32)
    zerosd = jnp.zeros((NPAD, D_OUT), jnp.float32)

    deg_parts = pl.kernel(
        _deg_body,
        out_type=jax.ShapeDtypeStruct((NC, NPAD, DEGW), jnp.float32),
        mesh=_MESH,
        scratch_types=[
            pltpu.VMEM((NCHUNK, K), jnp.int32),
            pltpu.VMEM((K, DEGW), jnp.float32),
            pltpu.VMEM_SHARED((NPAD, DEGW), jnp.float32),
        ],
    )(rows3, ones16, zeros16)

    grid = N // _R
    xw, y = pl.pallas_call(
        _pre_body,
        grid=(grid,),
        in_specs=[
            pl.BlockSpec((_R, D_IN), lambda i: (i, 0)),
            pl.BlockSpec((D_OUT, D_IN), lambda i: (0, 0)),
            pl.BlockSpec((NC, _R, DEGW), lambda i: (0, i, 0)),
        ],
        out_specs=[
            pl.BlockSpec((_R, D_OUT), lambda i: (i, 0)),
            pl.BlockSpec((_R, D_OUT), lambda i: (i, 0)),
        ],
        out_shape=[jax.ShapeDtypeStruct((N, D_OUT), jnp.float32)] * 2,
    )(x, W, deg_parts)

    acc_parts = pl.kernel(
        _edge_body,
        out_type=jax.ShapeDtypeStruct((NC, NPAD, D_OUT), jnp.float32),
        mesh=_MESH,
        scratch_types=[
            pltpu.VMEM((NCHUNK, K), jnp.int32),
            pltpu.VMEM((EW,), jnp.int32),
            pltpu.VMEM((K, D_OUT), jnp.float32),
            pltpu.VMEM((K, D_OUT), jnp.float32),
            pltpu.VMEM_SHARED((NPAD, D_OUT), jnp.float32),
            pltpu.SemaphoreType.DMA,
            pltpu.SemaphoreType.DMA,
            pltpu.SemaphoreType.DMA,
            pltpu.SemaphoreType.DMA,
        ],
    )(y, rows3, col, zerosd)

    out = pl.pallas_call(
        _post_body,
        grid=(grid,),
        in_specs=[
            pl.BlockSpec((NC, _R, D_OUT), lambda i: (0, i, 0)),
            pl.BlockSpec((NC, _R, DEGW), lambda i: (0, i, 0)),
            pl.BlockSpec((_R, D_OUT), lambda i: (i, 0)),
        ],
        out_specs=pl.BlockSpec((_R, D_OUT), lambda i: (i, 0)),
        out_shape=jax.ShapeDtypeStruct((N, D_OUT), jnp.float32),
    )(acc_parts, deg_parts, xw)
    return out


# speedup vs baseline: 1.0136x; 1.0136x over previous
import jax
import jax.numpy as jnp
from jax import lax
from jax.experimental import pallas as pl
from jax.experimental.pallas import tpu as pltpu
import jax.experimental.pallas.tpu_sc as plsc

N = 10000
E = 320000
D_IN = 128
D_OUT = 128

NC = 2
NS = 16
NW = NC * NS
EW = E // NW
K = 80
NCHUNK = EW // K
NPAD = 10240
RPT = NPAD // NS
DEGW = 16

_MESH = plsc.VectorSubcoreMesh(
    core_axis_name="c", subcore_axis_name="s", num_cores=NC, num_subcores=NS
)


def _deg_body(rows_hbm, ones_hbm, zeros_hbm, out_hbm, rowv, onesv, acc):
    c = lax.axis_index("c")
    s = lax.axis_index("s")
    g = c * NS + s
    pltpu.sync_copy(rows_hbm.at[g], rowv)
    pltpu.sync_copy(ones_hbm, onesv)
    sl = pl.ds(s * RPT, RPT)
    pltpu.sync_copy(zeros_hbm.at[sl], acc.at[sl])
    plsc.subcore_barrier()

    def step(j, carry):
        pltpu.sync_copy(onesv, acc.at[rowv.at[j]], add=True)
        return carry

    lax.fori_loop(0, NCHUNK, step, 0)
    plsc.subcore_barrier()
    pltpu.sync_copy(acc.at[sl], out_hbm.at[c, sl])


def _edge_body(y_hbm, rows_hbm, cols_hbm, zeros_hbm, out_hbm,
               rowv, colv, bufa, bufb, acc, sema, semb):
    c = lax.axis_index("c")
    s = lax.axis_index("s")
    g = c * NS + s
    pltpu.sync_copy(rows_hbm.at[g], rowv)
    pltpu.sync_copy(cols_hbm.at[pl.ds(g * EW, EW)], colv)
    sl = pl.ds(s * RPT, RPT)
    pltpu.sync_copy(zeros_hbm.at[sl], acc.at[sl])
    plsc.subcore_barrier()

    def cidx(j):
        return colv.at[pl.ds(j * K, K)]

    pltpu.async_copy(y_hbm.at[cidx(0)], bufa, sema)

    def pair(i, carry):
        ja = 2 * i
        jb = 2 * i + 1
        pltpu.async_copy(y_hbm.at[cidx(jb)], bufb, semb)
        pltpu.make_async_copy(y_hbm.at[cidx(ja)], bufa, sema).wait()
        pltpu.sync_copy(bufa, acc.at[rowv.at[ja]], add=True)
        pltpu.async_copy(y_hbm.at[cidx(jb + 1)], bufa, sema)
        pltpu.make_async_copy(y_hbm.at[cidx(jb)], bufb, semb).wait()
        pltpu.sync_copy(bufb, acc.at[rowv.at[jb]], add=True)
        return carry

    lax.fori_loop(0, (NCHUNK - 1) // 2, pair, 0)
    last = NCHUNK - 1
    pltpu.make_async_copy(y_hbm.at[cidx(last)], bufa, sema).wait()
    pltpu.sync_copy(bufa, acc.at[rowv.at[last]], add=True)

    plsc.subcore_barrier()
    pltpu.sync_copy(acc.at[sl], out_hbm.at[c, sl])


def _pre_body(x_ref, w_ref, dp_ref, xw_ref, y_ref):
    xw = lax.dot_general(
        x_ref[...], w_ref[...], (((1,), (1,)), ((), ())),
        preferred_element_type=jnp.float32)
    deg = dp_ref[0, :, 0:1] + dp_ref[1, :, 0:1] + 1.0
    xw_ref[...] = xw
    y_ref[...] = xw * (1.0 / jnp.sqrt(deg))


def _post_body(ap_ref, dp_ref, xw_ref, o_ref):
    deg = dp_ref[0, :, 0:1] + dp_ref[1, :, 0:1] + 1.0
    dis = 1.0 / jnp.sqrt(deg)
    o_ref[...] = dis * (ap_ref[0] + ap_ref[1]) + xw_ref[...] / deg


_R = 400


@jax.jit
def kernel(x, edge_index, W):
    row = edge_index[0].astype(jnp.int32)
    col = edge_index[1].astype(jnp.int32)
    rows3 = row.reshape(NW, NCHUNK, K)
    ones16 = jnp.ones((K, DEGW), jnp.float32)
    zeros16 = jnp.zeros((NPAD, DEGW), jnp.float32)
    zerosd = jnp.zeros((NPAD, D_OUT), jnp.float32)

    deg_parts = pl.kernel(
        _deg_body,
        out_type=jax.ShapeDtypeStruct((NC, NPAD, DEGW), jnp.float32),
        mesh=_MESH,
        scratch_types=[
            pltpu.VMEM((NCHUNK, K), jnp.int32),
            pltpu.VMEM((K, DEGW), jnp.float32),
            pltpu.VMEM_SHARED((NPAD, DEGW), jnp.float32),
        ],
    )(rows3, ones16, zeros16)

    grid = N // _R
    xw, y = pl.pallas_call(
        _pre_body,
        grid=(grid,),
        in_specs=[
            pl.BlockSpec((_R, D_IN), lambda i: (i, 0)),
            pl.BlockSpec((D_OUT, D_IN), lambda i: (0, 0)),
            pl.BlockSpec((NC, _R, DEGW), lambda i: (0, i, 0)),
        ],
        out_specs=[
            pl.BlockSpec((_R, D_OUT), lambda i: (i, 0)),
            pl.BlockSpec((_R, D_OUT), lambda i: (i, 0)),
        ],
        out_shape=[jax.ShapeDtypeStruct((N, D_OUT), jnp.float32)] * 2,
    )(x, W, deg_parts)

    acc_parts = pl.kernel(
        _edge_body,
        out_type=jax.ShapeDtypeStruct((NC, NPAD, D_OUT), jnp.float32),
        mesh=_MESH,
        scratch_types=[
            pltpu.VMEM((NCHUNK, K), jnp.int32),
            pltpu.VMEM((EW,), jnp.int32),
            pltpu.VMEM((K, D_OUT), jnp.float32),
            pltpu.VMEM((K, D_OUT), jnp.float32),
            pltpu.VMEM_SHARED((NPAD, D_OUT), jnp.float32),
            pltpu.SemaphoreType.DMA,
            pltpu.SemaphoreType.DMA,
        ],
    )(y, rows3, col, zerosd)

    out = pl.pallas_call(
        _post_body,
        grid=(grid,),
        in_specs=[
            pl.BlockSpec((NC, _R, D_OUT), lambda i: (0, i, 0)),
            pl.BlockSpec((NC, _R, DEGW), lambda i: (0, i, 0)),
            pl.BlockSpec((_R, D_OUT), lambda i: (i, 0)),
        ],
        out_specs=pl.BlockSpec((_R, D_OUT), lambda i: (i, 0)),
        out_shape=jax.ShapeDtypeStruct((N, D_OUT), jnp.float32),
    )(acc_parts, deg_parts, xw)
    return out
